# Initial kernel scaffold; baseline (speedup 1.0000x reference)
import jax
import jax.numpy as jnp
from jax.experimental import pallas as pl


def kernel(anchors, objectness, box_regression):
    raise NotImplementedError("write your pallas kernel here")


# speedup vs baseline: 2.6437x; 2.6437x over previous
import functools

import jax
import jax.numpy as jnp
import numpy as np
from jax.experimental import pallas as pl
from jax.experimental.pallas import tpu as pltpu

_N, _A, _H, _W = 2, 3, 160, 160
_NANCH = _A * _H * _W
_ROWS, _LANES = _NANCH // 128, 128
_K = 6000
_POST = 1000
_NMS_THRESH = 0.7
_IM_W, _IM_H = 800.0, 800.0
_BBOX_CLIP = float(np.log(1000.0 / 16.0))
_ONE_BITS = 0x3F800000


def _nms_body(s_ref, ax1_ref, ay1_ref, ax2_ref, ay2_ref,
              rx_ref, ry_ref, rw_ref, rh_ref,
              out_ref,
              bx1_s, by1_s, bx2_s, by2_s, bar_s, sc_s):
    f32 = jnp.float32
    i32 = jnp.int32

    s = s_ref[0]
    si = jax.lax.bitcast_convert_type(s, i32)
    r_iota = jax.lax.broadcasted_iota(i32, (_ROWS, _LANES), 0)
    l_iota = jax.lax.broadcasted_iota(i32, (_ROWS, _LANES), 1)
    flat_iota = r_iota * _LANES + l_iota

    def _tau_step(_, lohi):
        lo, hi = lohi
        mid = lo + (hi - lo + 1) // 2
        cnt = jnp.sum((si >= mid).astype(i32))
        big = cnt >= _K
        return (jnp.where(big, mid, lo), jnp.where(big, hi, mid - 1))

    tau, _ = jax.lax.fori_loop(0, 31, _tau_step,
                               (jnp.int32(0), jnp.int32(_ONE_BITS)))
    n_gt = jnp.sum((si > tau).astype(i32))
    needed = _K - n_gt
    eq = si == tau

    def _m_step(_, lohi):
        lo, hi = lohi
        mid = (lo + hi) // 2
        cnt = jnp.sum((eq & (flat_iota < mid)).astype(i32))
        ok = cnt >= needed
        return (jnp.where(ok, lo, mid + 1), jnp.where(ok, mid, hi))

    m_cut, _ = jax.lax.fori_loop(0, 18, _m_step,
                                 (jnp.int32(0), jnp.int32(_NANCH)))
    in_set = (si > tau) | (eq & (flat_iota < m_cut))
    sc_s[:] = jnp.where(in_set, s, f32(-1.0))

    ax1, ay1, ax2, ay2 = ax1_ref[0], ay1_ref[0], ax2_ref[0], ay2_ref[0]
    widths = ax2 - ax1 + 1.0
    heights = ay2 - ay1 + 1.0
    ctr_x = ax1 + 0.5 * widths
    ctr_y = ay1 + 0.5 * heights
    dx, dy = rx_ref[0], ry_ref[0]
    dw = jnp.minimum(rw_ref[0], _BBOX_CLIP)
    dh = jnp.minimum(rh_ref[0], _BBOX_CLIP)
    pred_ctr_x = dx * widths + ctr_x
    pred_ctr_y = dy * heights + ctr_y
    pred_w = jnp.exp(dw) * widths
    pred_h = jnp.exp(dh) * heights
    x1 = jnp.clip(pred_ctr_x - 0.5 * pred_w, 0.0, _IM_W - 1.0)
    y1 = jnp.clip(pred_ctr_y - 0.5 * pred_h, 0.0, _IM_H - 1.0)
    x2 = jnp.clip(pred_ctr_x + 0.5 * pred_w - 1.0, 0.0, _IM_W - 1.0)
    y2 = jnp.clip(pred_ctr_y + 0.5 * pred_h - 1.0, 0.0, _IM_H - 1.0)
    bx1_s[:] = x1
    by1_s[:] = y1
    bx2_s[:] = x2
    by2_s[:] = y2
    bar_s[:] = (x2 - x1 + 1.0) * (y2 - y1 + 1.0)

    lane1 = jax.lax.broadcasted_iota(i32, (1, _LANES), 1)
    iota8 = jax.lax.broadcasted_iota(i32, (1, 8), 1)

    def _step(t, _):
        s_cur = sc_s[:]
        m = jnp.max(s_cur)
        fi = jnp.min(jnp.where(s_cur == m, flat_iota, _NANCH))
        valid = m > 0.0
        r = fi // _LANES
        c = fi % _LANES
        lane_eq = lane1 == c

        def _pick(ref):
            row = ref[pl.ds(r, 1), :]
            return jnp.sum(jnp.where(lane_eq, row, f32(0.0)))

        bx1 = _pick(bx1_s)
        by1 = _pick(by1_s)
        bx2 = _pick(bx2_s)
        by2 = _pick(by2_s)
        bar = _pick(bar_s)

        xx1 = jnp.maximum(bx1, bx1_s[:])
        yy1 = jnp.maximum(by1, by1_s[:])
        xx2 = jnp.minimum(bx2, bx2_s[:])
        yy2 = jnp.minimum(by2, by2_s[:])
        w = jnp.maximum(xx2 - xx1 + 1.0, 0.0)
        h = jnp.maximum(yy2 - yy1 + 1.0, 0.0)
        inter = w * h
        iou = inter / (bar + bar_s[:] - inter)
        supp = (iou > _NMS_THRESH) | (flat_iota == fi)
        sc_s[:] = jnp.where(supp & valid, f32(-1.0), s_cur)

        vals = (bx1, by1, bx2, by2, m, f32(1.0))
        row8 = jnp.zeros((1, 8), f32)
        for j, v in enumerate(vals):
            row8 = jnp.where(iota8 == j, v, row8)
        out_ref[0, pl.ds(t, 1), :] = jnp.where(valid, row8, f32(0.0))
        return 0

    jax.lax.fori_loop(0, _POST, _step, 0)


@jax.jit
def kernel(anchors, objectness, box_regression):
    obj = jnp.transpose(objectness, (0, 2, 3, 1)).reshape(_N, _NANCH)
    scores = jax.nn.sigmoid(obj).reshape(_N, _ROWS, _LANES)
    reg = box_regression.reshape(_N, _A, 4, _H, _W)
    reg = jnp.transpose(reg, (0, 3, 4, 1, 2)).reshape(_N, _NANCH, 4)
    anc = anchors.reshape(_N, _NANCH, 4)

    def _planes(arr):
        return tuple(arr[..., j].reshape(_N, _ROWS, _LANES) for j in range(4))

    ax1, ay1, ax2, ay2 = _planes(anc)
    rx, ry, rw, rh = _planes(reg)

    blk = pl.BlockSpec((1, _ROWS, _LANES), lambda n: (n, 0, 0))
    out = pl.pallas_call(
        _nms_body,
        grid=(_N,),
        in_specs=[blk] * 9,
        out_specs=pl.BlockSpec((1, _POST, 8), lambda n: (n, 0, 0)),
        out_shape=jax.ShapeDtypeStruct((_N, _POST, 8), jnp.float32),
        scratch_shapes=[pltpu.VMEM((_ROWS, _LANES), jnp.float32)] * 6,
    )(scores, ax1, ay1, ax2, ay2, rx, ry, rw, rh)

    return out[..., 0:4], out[..., 4], out[..., 5]


# speedup vs baseline: 3.2753x; 1.2389x over previous
import functools

import jax
import jax.numpy as jnp
import numpy as np
from jax import lax
from jax.experimental import pallas as pl
from jax.experimental.pallas import tpu as pltpu
from jax.experimental.pallas import tpu_sc as plsc

_N, _A, _H, _W = 2, 3, 160, 160
_NANCH = _A * _H * _W
_ROWS, _LANES = _NANCH // 128, 128
_K = 6000
_POST = 1000
_NMS_THRESH = 0.7
_IM_W, _IM_H = 800.0, 800.0
_BBOX_CLIP = float(np.log(1000.0 / 16.0))
_ONE_BITS = 0x3F800000

_NSUB = 16
_CHUNK = _NANCH // _NSUB
_SLICES = _CHUNK // 16
_CROWS = 50
_CPAD = _CROWS * _LANES
_MAXCH = (_CHUNK + 127) // 128


def _prep_body(s_ref, ax1_ref, ay1_ref, ax2_ref, ay2_ref,
               rx_ref, ry_ref, rw_ref, rh_ref,
               x1_o, y1_o, x2_o, y2_o, meta_o):
    f32 = jnp.float32
    i32 = jnp.int32

    s = s_ref[0]
    si = jax.lax.bitcast_convert_type(s, i32)
    r_iota = jax.lax.broadcasted_iota(i32, (_ROWS, _LANES), 0)
    l_iota = jax.lax.broadcasted_iota(i32, (_ROWS, _LANES), 1)
    flat_iota = r_iota * _LANES + l_iota

    def _tau_step(_, lohi):
        lo, hi = lohi
        mid = lo + (hi - lo + 1) // 2
        cnt = jnp.sum((si >= mid).astype(i32))
        big = cnt >= _K
        return (jnp.where(big, mid, lo), jnp.where(big, hi, mid - 1))

    tau, _ = jax.lax.fori_loop(0, 31, _tau_step,
                               (jnp.int32(0), jnp.int32(_ONE_BITS)))
    n_gt = jnp.sum((si > tau).astype(i32))
    needed = _K - n_gt
    eq = si == tau

    def _m_step(_, lohi):
        lo, hi = lohi
        mid = (lo + hi) // 2
        cnt = jnp.sum((eq & (flat_iota < mid)).astype(i32))
        ok = cnt >= needed
        return (jnp.where(ok, lo, mid + 1), jnp.where(ok, mid, hi))

    m_cut, _ = jax.lax.fori_loop(0, 18, _m_step,
                                 (jnp.int32(0), jnp.int32(_NANCH)))
    in_set = (si > tau) | (eq & (flat_iota < m_cut))

    chunk_id = flat_iota // _CHUNK
    lane_row = jax.lax.broadcasted_iota(i32, (1, _LANES), 1)
    counts = jnp.zeros((1, _LANES), i32)
    for k in range(_NSUB):
        ck = jnp.sum((in_set & (chunk_id == k)).astype(i32))
        counts = jnp.where(lane_row == k, ck, counts)
    meta_o[0, pl.ds(0, 1), :] = counts
    meta_o[0, pl.ds(1, 1), :] = jnp.full((1, _LANES), tau, i32)
    meta_o[0, pl.ds(2, 1), :] = jnp.full((1, _LANES), m_cut, i32)

    ax1, ay1, ax2, ay2 = ax1_ref[0], ay1_ref[0], ax2_ref[0], ay2_ref[0]
    widths = ax2 - ax1 + 1.0
    heights = ay2 - ay1 + 1.0
    ctr_x = ax1 + 0.5 * widths
    ctr_y = ay1 + 0.5 * heights
    dx, dy = rx_ref[0], ry_ref[0]
    dw = jnp.minimum(rw_ref[0], _BBOX_CLIP)
    dh = jnp.minimum(rh_ref[0], _BBOX_CLIP)
    pred_ctr_x = dx * widths + ctr_x
    pred_ctr_y = dy * heights + ctr_y
    pred_w = jnp.exp(dw) * widths
    pred_h = jnp.exp(dh) * heights
    x1_o[0] = jnp.clip(pred_ctr_x - 0.5 * pred_w, 0.0, _IM_W - 1.0)
    y1_o[0] = jnp.clip(pred_ctr_y - 0.5 * pred_h, 0.0, _IM_H - 1.0)
    x2_o[0] = jnp.clip(pred_ctr_x + 0.5 * pred_w - 1.0, 0.0, _IM_W - 1.0)
    y2_o[0] = jnp.clip(pred_ctr_y + 0.5 * pred_h - 1.0, 0.0, _IM_H - 1.0)


_sc_mesh = plsc.VectorSubcoreMesh(core_axis_name="c", subcore_axis_name="s")


@functools.partial(
    pl.kernel,
    mesh=_sc_mesh,
    compiler_params=pltpu.CompilerParams(needs_layout_passes=False),
    out_type=[jax.ShapeDtypeStruct((_N * _CPAD,), jnp.float32)] * 5,
    scratch_types=[
        pltpu.VMEM((_CHUNK,), jnp.float32),
        pltpu.VMEM((16,), jnp.int32),
        pltpu.VMEM((16,), jnp.int32),
        pltpu.VMEM((16,), jnp.int32),
        pltpu.VMEM((_MAXCH * 128,), jnp.int32),
        pltpu.VMEM((_MAXCH, 128), jnp.int32),
        pltpu.VMEM((128,), jnp.float32),
        pltpu.VMEM((128,), jnp.float32),
        pltpu.VMEM((128,), jnp.float32),
        pltpu.VMEM((128,), jnp.float32),
        pltpu.VMEM((128,), jnp.float32),
        pltpu.SemaphoreType.DMA,
        pltpu.SemaphoreType.DMA,
    ],
)
def _sc_compact(scores_hbm, x1_hbm, y1_hbm, x2_hbm, y2_hbm, meta_hbm,
                ox1, oy1, ox2, oy2, osc,
                sc_v, cnt_v, tau_v, mcut_v, idxg, idxs,
                g0, g1, g2, g3, g4, sem_g, sem_s):
    i32 = jnp.int32
    img = lax.axis_index("c")
    sid = lax.axis_index("s")
    lane = lax.iota(i32, 16)

    base = pl.multiple_of(img * _NANCH + sid * _CHUNK, 8)
    pltpu.sync_copy(scores_hbm.at[pl.ds(base, _CHUNK)], sc_v)
    mbase = pl.multiple_of(img * 3 * _LANES, 8)
    pltpu.sync_copy(meta_hbm.at[pl.ds(mbase, 16)], cnt_v)
    pltpu.sync_copy(meta_hbm.at[pl.ds(mbase + _LANES, 16)], tau_v)
    pltpu.sync_copy(meta_hbm.at[pl.ds(mbase + 2 * _LANES, 16)], mcut_v)

    cnts = cnt_v[...]
    tau = tau_v[...]
    mcut = mcut_v[...]
    my_off = jnp.sum(jnp.where(lane < sid, cnts, 0))
    my_cnt = jnp.sum(jnp.where(lane == sid, cnts, 0))

    trash_g = jnp.full((16,), _N * _NANCH - 16, i32) + lane
    trash_s = img * _CPAD + 6144 + lane

    def _pref(j, _):
        idxg[pl.ds(j * 16, 16)] = trash_g
        idxs[j // 8, pl.ds((j % 8) * 16, 16)] = trash_s
        return 0

    lax.fori_loop(0, _MAXCH * 8, _pref, 0)

    cbase = sid * _CHUNK

    def _cstep(j, w):
        sv = sc_v[pl.ds(j * 16, 16)]
        si = lax.bitcast_convert_type(sv, i32)
        gidx = cbase + j * 16 + lane
        mask = (si > tau) | ((si == tau) & (gidx < mcut))
        mi = mask.astype(i32)
        cum = plsc.cumsum(mi)
        pos = w + cum - 1
        plsc.store_scatter(idxg, [pos], gidx + img * _NANCH, mask=mask)
        plsc.store_scatter(idxs, [pos // 128, pos % 128],
                           img * _CPAD + my_off + pos, mask=mask)
        return w + jnp.sum(mi)

    lax.fori_loop(0, _SLICES, _cstep, jnp.int32(0))

    nch = (my_cnt + 127) // 128

    def _dma(j, _):
        s0 = j * 128
        gs = idxg.at[pl.ds(s0, 128)]
        hs = [pltpu.async_copy(src.at[gs], dst, sem_g)
              for src, dst in ((x1_hbm, g0), (y1_hbm, g1), (x2_hbm, g2),
                               (y2_hbm, g3), (scores_hbm, g4))]
        for h in hs:
            h.wait()
        ss = idxs.at[j]
        ks = [pltpu.async_copy(src, dst.at[ss], sem_s)
              for src, dst in ((g0, ox1), (g1, oy1), (g2, ox2),
                               (g3, oy2), (g4, osc))]
        for k in ks:
            k.wait()
        return 0

    lax.fori_loop(0, nch, _dma, 0)


def _nms_body(x1_ref, y1_ref, x2_ref, y2_ref, s_ref, out_ref, sc_s, bar_s):
    f32 = jnp.float32
    i32 = jnp.int32

    r_iota = jax.lax.broadcasted_iota(i32, (_CROWS, _LANES), 0)
    l_iota = jax.lax.broadcasted_iota(i32, (_CROWS, _LANES), 1)
    flat_iota = r_iota * _LANES + l_iota

    sc_s[:] = jnp.where(flat_iota < _K, s_ref[0], f32(-1.0))
    x1a, y1a = x1_ref[0], y1_ref[0]
    x2a, y2a = x2_ref[0], y2_ref[0]
    bar_s[:] = (x2a - x1a + 1.0) * (y2a - y1a + 1.0)

    lane1 = jax.lax.broadcasted_iota(i32, (1, _LANES), 1)
    iota8 = jax.lax.broadcasted_iota(i32, (1, 8), 1)

    def _step(t, _):
        s_cur = sc_s[:]
        m = jnp.max(s_cur)
        fi = jnp.min(jnp.where(s_cur == m, flat_iota, _CPAD))
        valid = m > 0.0
        r = fi // _LANES
        c = fi % _LANES
        lane_eq = lane1 == c

        def _pick(ref):
            row = ref[0, pl.ds(r, 1), :]
            return jnp.sum(jnp.where(lane_eq, row, f32(0.0)))

        bx1 = _pick(x1_ref)
        by1 = _pick(y1_ref)
        bx2 = _pick(x2_ref)
        by2 = _pick(y2_ref)
        bar = jnp.sum(jnp.where(lane_eq, bar_s[pl.ds(r, 1), :], f32(0.0)))

        xx1 = jnp.maximum(bx1, x1a)
        yy1 = jnp.maximum(by1, y1a)
        xx2 = jnp.minimum(bx2, x2a)
        yy2 = jnp.minimum(by2, y2a)
        w = jnp.maximum(xx2 - xx1 + 1.0, 0.0)
        h = jnp.maximum(yy2 - yy1 + 1.0, 0.0)
        inter = w * h
        iou = inter / (bar + bar_s[:] - inter)
        supp = (iou > _NMS_THRESH) | (flat_iota == fi)
        sc_s[:] = jnp.where(supp & valid, f32(-1.0), s_cur)

        vals = (bx1, by1, bx2, by2, m, f32(1.0))
        row8 = jnp.zeros((1, 8), f32)
        for j, v in enumerate(vals):
            row8 = jnp.where(iota8 == j, v, row8)
        out_ref[0, pl.ds(t, 1), :] = jnp.where(valid, row8, f32(0.0))
        return 0

    jax.lax.fori_loop(0, _POST, _step, 0)


@jax.jit
def kernel(anchors, objectness, box_regression):
    obj = jnp.transpose(objectness, (0, 2, 3, 1)).reshape(_N, _NANCH)
    scores = jax.nn.sigmoid(obj).reshape(_N, _ROWS, _LANES)
    reg = box_regression.reshape(_N, _A, 4, _H, _W)
    reg = jnp.transpose(reg, (0, 3, 4, 1, 2)).reshape(_N, _NANCH, 4)
    anc = anchors.reshape(_N, _NANCH, 4)

    def _planes(arr):
        return tuple(arr[..., j].reshape(_N, _ROWS, _LANES) for j in range(4))

    ax1, ay1, ax2, ay2 = _planes(anc)
    rx, ry, rw, rh = _planes(reg)

    blk = pl.BlockSpec((1, _ROWS, _LANES), lambda n: (n, 0, 0))
    x1p, y1p, x2p, y2p, meta = pl.pallas_call(
        _prep_body,
        grid=(_N,),
        in_specs=[blk] * 9,
        out_specs=[blk] * 4 + [pl.BlockSpec((1, 3, _LANES), lambda n: (n, 0, 0))],
        out_shape=[jax.ShapeDtypeStruct((_N, _ROWS, _LANES), jnp.float32)] * 4
        + [jax.ShapeDtypeStruct((_N, 3, _LANES), jnp.int32)],
    )(scores, ax1, ay1, ax2, ay2, rx, ry, rw, rh)

    flat = lambda a: a.reshape(-1)
    cx1, cy1, cx2, cy2, csc = _sc_compact(
        flat(scores), flat(x1p), flat(y1p), flat(x2p), flat(y2p), flat(meta))

    cshape = lambda a: a.reshape(_N, _CROWS, _LANES)
    cblk = pl.BlockSpec((1, _CROWS, _LANES), lambda n: (n, 0, 0))
    out = pl.pallas_call(
        _nms_body,
        grid=(_N,),
        in_specs=[cblk] * 5,
        out_specs=pl.BlockSpec((1, _POST, 8), lambda n: (n, 0, 0)),
        out_shape=jax.ShapeDtypeStruct((_N, _POST, 8), jnp.float32),
        scratch_shapes=[pltpu.VMEM((_CROWS, _LANES), jnp.float32)] * 2,
    )(cshape(cx1), cshape(cy1), cshape(cx2), cshape(cy2), cshape(csc))

    return out[..., 0:4], out[..., 4], out[..., 5]
